# Initial kernel scaffold; baseline (speedup 1.0000x reference)
import jax
import jax.numpy as jnp
from jax.experimental import pallas as pl


def kernel(user_id, prev_item_id, time_since_prev_item, item_id, time_since_prev_user, dynamic_user_emb, dynamic_item_emb, is_user_new, is_item_new, static_user_table, static_item_table, initial_user_emb, initial_item_emb, user_W_ih, user_b_ih, user_W_hh, user_b_hh, item_W_ih, item_b_ih, item_W_hh, item_b_hh, pred_W, pred_b, td_W, td_b):
    raise NotImplementedError("write your pallas kernel here")


# speedup vs baseline: 1.5804x; 1.5804x over previous
import functools

import jax
import jax.numpy as jnp
from jax import lax
from jax.experimental import pallas as pl
from jax.experimental.pallas import tpu as pltpu
from jax.experimental.pallas import tpu_sc as plsc

NUM_USERS = 1000000
NUM_ITEMS = 100000
D = 64
B = 16384

NC = 2
NS = 16
NW = NC * NS
CH = 128
PB = B // NW
NSUB = PB // CH
ROWS_ID = B // CH
RPW = ROWS_ID // NW

@functools.cache
def _mesh():
  return plsc.VectorSubcoreMesh(
      core_axis_name="c", subcore_axis_name="s", num_cores=NC, num_subcores=NS)


def _wid():
  return lax.axis_index("s") * NC + lax.axis_index("c")


def _gather_body(du_t, di_t, su_t, si_t, fu_t, fi_t, uid, iid, pid,
                 du_o, di_o, dpi_o, su_o, si_o, spi_o, fu_o, fi_o, fpi_o,
                 idx_u, idx_i, idx_p, rows, flg, sem):
  w = _wid()
  base_r = w * RPW
  pltpu.sync_copy(uid.at[pl.ds(base_r, RPW)], idx_u)
  pltpu.sync_copy(iid.at[pl.ds(base_r, RPW)], idx_i)
  pltpu.sync_copy(pid.at[pl.ds(base_r, RPW)], idx_p)
  base = w * PB
  for tbl, idx, out in ((du_t, idx_u, du_o), (di_t, idx_i, di_o),
                        (di_t, idx_p, dpi_o), (su_t, idx_u, su_o),
                        (si_t, idx_i, si_o), (si_t, idx_p, spi_o)):
    for j in range(NSUB):
      pltpu.async_copy(tbl.at[idx.at[j]], rows, sem).wait()
      pltpu.sync_copy(rows, out.at[pl.ds(base + j * CH, CH)])
  for tbl, idx, out in ((fu_t, idx_u, fu_o), (fi_t, idx_i, fi_o),
                        (fi_t, idx_p, fpi_o)):
    for j in range(NSUB):
      pltpu.async_copy(tbl.at[idx.at[j]], flg, sem).wait()
      pltpu.sync_copy(flg, out.at[pl.ds(base + j * CH, CH)])


@functools.cache
def _gather():
  return pl.kernel(
      _gather_body,
      out_type=[jax.ShapeDtypeStruct((B, D), jnp.float32)] * 6
      + [jax.ShapeDtypeStruct((B,), jnp.float32)] * 3,
      mesh=_mesh(),
      compiler_params=pltpu.CompilerParams(use_tc_tiling_on_sc=False,
                                           needs_layout_passes=False),
      scratch_types=[
          pltpu.VMEM((RPW, CH), jnp.int32),
          pltpu.VMEM((RPW, CH), jnp.int32),
          pltpu.VMEM((RPW, CH), jnp.int32),
          pltpu.VMEM((CH, D), jnp.float32),
          pltpu.VMEM((CH,), jnp.float32),
          pltpu.SemaphoreType.DMA,
      ],
  )


BLK = 1024


def _dense_body(du, di, dpi, su, si, spi, fu, fi, fpi, ti, tu,
                uWe, uwt, ubih, uWh, ubhh, iWe, iwt, ibih, iWh, ibhh,
                predT, pb, tdw, tdb, iu_e, ii_e,
                pred_o, tgt_o, uu_o, ue_o, ui_o, ie_o):
  f32 = jnp.float32
  ue = fu[...] * iu_e[...] + du[...]
  ie = fi[...] * ii_e[...] + di[...]
  pe = fpi[...] * ii_e[...] + dpi[...]
  td = ti[...] * tdw[...] + tdb[...]
  up = ue * (1.0 + td)
  pT = predT[...]
  pred = (jnp.dot(up, pT[0:D], preferred_element_type=f32)
          + jnp.dot(pe, pT[D:2 * D], preferred_element_type=f32)
          + jnp.dot(spi[...], pT[2 * D:3 * D], preferred_element_type=f32)
          + jnp.dot(su[...], pT[3 * D:4 * D], preferred_element_type=f32)
          + pb[...])
  pred_o[...] = pred
  tgt_o[:, 0:D] = ie
  tgt_o[:, D:2 * D] = si[...]
  uu = jnp.tanh(jnp.dot(ie, uWe[...], preferred_element_type=f32)
                + ti[...] * uwt[...] + ubih[...]
                + jnp.dot(ue, uWh[...], preferred_element_type=f32)
                + ubhh[...])
  ui = jnp.tanh(jnp.dot(ue, iWe[...], preferred_element_type=f32)
                + tu[...] * iwt[...] + ibih[...]
                + jnp.dot(ie, iWh[...], preferred_element_type=f32)
                + ibhh[...])
  uu_o[...] = uu
  ue_o[...] = ue
  ui_o[...] = ui
  ie_o[...] = ie


def _dense(du, di, dpi, su, si, spi, fu, fi, fpi, ti, tu,
           uWe, uwt, ubih, uWh, ubhh, iWe, iwt, ibih, iWh, ibhh,
           predT, pb, tdw, tdb, iu_e, ii_e):
  nblk = B // BLK
  row = lambda i: (i, 0)
  fix = lambda i: (0, 0)
  bspec = lambda shp, im: pl.BlockSpec(shp, im)
  in_specs = (
      [bspec((BLK, D), row)] * 6 + [bspec((BLK, 1), row)] * 5
      + [bspec((D, D), fix), bspec((1, D), fix), bspec((1, D), fix),
         bspec((D, D), fix), bspec((1, D), fix)] * 2
      + [bspec((4 * D, 2 * D), fix), bspec((1, 2 * D), fix),
         bspec((1, D), fix), bspec((1, D), fix),
         bspec((1, D), fix), bspec((1, D), fix)]
  )
  out_specs = [bspec((BLK, 2 * D), row), bspec((BLK, 2 * D), row),
               bspec((BLK, D), row), bspec((BLK, D), row),
               bspec((BLK, D), row), bspec((BLK, D), row)]
  out_shape = [jax.ShapeDtypeStruct((B, 2 * D), jnp.float32),
               jax.ShapeDtypeStruct((B, 2 * D), jnp.float32),
               jax.ShapeDtypeStruct((B, D), jnp.float32),
               jax.ShapeDtypeStruct((B, D), jnp.float32),
               jax.ShapeDtypeStruct((B, D), jnp.float32),
               jax.ShapeDtypeStruct((B, D), jnp.float32)]
  return pl.pallas_call(
      _dense_body,
      grid=(nblk,),
      in_specs=in_specs,
      out_specs=out_specs,
      out_shape=out_shape,
  )(du, di, dpi, su, si, spi, fu, fi, fpi, ti, tu,
    uWe, uwt, ubih, uWh, ubhh, iWe, iwt, ibih, iWh, ibhh,
    predT, pb, tdw, tdb, iu_e, ii_e)


NVREG = B // 16
AUXN = (NUM_USERS + NW - 1) // NW + 32


def _process(w, ids_hbm, vals_hbm, table, flag_t,
             ids, aux, blist, bwin, idwin, rows, zer, sem):
  pltpu.sync_copy(ids_hbm, ids)
  iota = lax.iota(jnp.int32, 16)

  def phase_a(j, off):
    idv = ids[pl.ds(j * 16, 16)]
    bv = j * 16 + iota
    m = (idv & (NW - 1)) == w
    pos = plsc.cumsum(m.astype(jnp.int32))
    dest = off + pos - 1
    plsc.store_scatter(blist, [dest], bv, mask=m)
    return off + plsc.all_reduce_population_count(m)

  off = lax.fori_loop(0, NVREG, phase_a, jnp.zeros((16,), jnp.int32))
  cnt = jnp.max(off)
  nv = (cnt + 15) >> 4

  def phase_b(k0, _):
    kv = k0 * 16 + iota
    valid = kv < cnt
    bv = blist[pl.ds(k0 * 16, 16)]
    idvv = plsc.load_gather(ids, [bv], mask=valid)
    slot = idvv >> 5
    _, lastm = plsc.scan_count(slot, valid)
    plsc.store_scatter(aux, [slot], kv, mask=valid & lastm)
    return 0

  lax.fori_loop(0, nv, phase_b, 0)

  def phase_c(k0, carry):
    woff, lastb, lastid = carry
    kv = k0 * 16 + iota
    valid = kv < cnt
    bv = blist[pl.ds(k0 * 16, 16)]
    idvv = plsc.load_gather(ids, [bv], mask=valid)
    av = plsc.load_gather(aux, [idvv >> 5], mask=valid)
    win = valid & (av == kv)
    wd = woff + plsc.cumsum(win.astype(jnp.int32)) - 1
    plsc.store_scatter(bwin, [wd >> 7, wd & (CH - 1)], bv, mask=win)
    plsc.store_scatter(idwin, [wd >> 7, wd & (CH - 1)], idvv, mask=win)
    lane = lax.iota(jnp.int32, 16)
    mb = jnp.max(jnp.where(win, bv, -1))
    combo = jnp.max(jnp.where(win, (lane << 20) | idvv, -1))
    lastb = jnp.where(mb >= 0, mb, lastb)
    lastid = jnp.where(combo >= 0, combo & 0xFFFFF, lastid)
    return (woff + plsc.all_reduce_population_count(win), lastb, lastid)

  woff, lastb, lastid = lax.fori_loop(
      0, nv, phase_c, (jnp.zeros((16,), jnp.int32),
                       jnp.int32(0), jnp.int32(0)))
  wcnt = jnp.max(woff)

  lastb_v = jnp.full((16,), lastb, jnp.int32)
  lastid_v = jnp.full((16,), lastid, jnp.int32)

  def pad(j, _):
    kv = j * 16 + iota
    needpad = kv >= wcnt
    plsc.store_scatter(bwin, [kv >> 7, kv & (CH - 1)], lastb_v, mask=needpad)
    plsc.store_scatter(idwin, [kv >> 7, kv & (CH - 1)], lastid_v,
                       mask=needpad)
    return 0

  lax.fori_loop(wcnt >> 4, ((wcnt + CH - 1) >> 7) << 3, pad, 0)

  def phase_d(c, _):
    pltpu.async_copy(vals_hbm.at[bwin.at[c]], rows, sem).wait()
    pltpu.async_copy(rows, table.at[idwin.at[c]], sem).wait()
    pltpu.async_copy(zer, flag_t.at[idwin.at[c]], sem).wait()
    return 0

  lax.fori_loop(0, (wcnt + CH - 1) >> 7, phase_d, 0)


def _scatter_body(du_t, di_t, fu_t, fi_t, uid1, iid1, uu, ui,
                  ids, aux, blist, bwin, idwin, rows, zer, sem):
  w = _wid()
  for i in range(8):
    zer[pl.ds(i * 16, 16)] = jnp.zeros((16,), jnp.float32)
  _process(w, uid1, uu, du_t, fu_t,
           ids, aux, blist, bwin, idwin, rows, zer, sem)
  _process(w, iid1, ui, di_t, fi_t,
           ids, aux, blist, bwin, idwin, rows, zer, sem)


@functools.cache
def _scatter():
  return pl.kernel(
      _scatter_body,
      out_type=(),
      mesh=_mesh(),
      compiler_params=pltpu.CompilerParams(use_tc_tiling_on_sc=False,
                                           needs_layout_passes=False),
      scratch_types=[
          pltpu.VMEM((B,), jnp.int32),
          pltpu.VMEM((AUXN,), jnp.int32),
          pltpu.VMEM((B,), jnp.int32),
          pltpu.VMEM((B // CH, CH), jnp.int32),
          pltpu.VMEM((B // CH, CH), jnp.int32),
          pltpu.VMEM((CH, D), jnp.float32),
          pltpu.VMEM((CH,), jnp.float32),
          pltpu.SemaphoreType.DMA,
      ],
  )


def kernel(user_id, prev_item_id, time_since_prev_item, item_id,
           time_since_prev_user, dynamic_user_emb, dynamic_item_emb,
           is_user_new, is_item_new, static_user_table, static_item_table,
           initial_user_emb, initial_item_emb,
           user_W_ih, user_b_ih, user_W_hh, user_b_hh,
           item_W_ih, item_b_ih, item_W_hh, item_b_hh,
           pred_W, pred_b, td_W, td_b):
  uid2 = user_id.reshape(ROWS_ID, CH)
  iid2 = item_id.reshape(ROWS_ID, CH)
  pid2 = prev_item_id.reshape(ROWS_ID, CH)
  fu1 = is_user_new.reshape(-1)
  fi1 = is_item_new.reshape(-1)

  du, di, dpi, su, si, spi, fu, fi, fpi = _gather()(
      dynamic_user_emb, dynamic_item_emb, static_user_table,
      static_item_table, fu1, fi1, uid2, iid2, pid2)

  uWe = user_W_ih[:, :D].T
  uwt = user_W_ih[:, D].reshape(1, D)
  uWh = user_W_hh.T
  iWe = item_W_ih[:, :D].T
  iwt = item_W_ih[:, D].reshape(1, D)
  iWh = item_W_hh.T
  predT = pred_W.T
  pb = pred_b.reshape(1, 2 * D)
  tdw = td_W.reshape(1, D)
  tdb = td_b.reshape(1, D)

  item_pred, item_target, uu, ue, ui, ie = _dense(
      du, di, dpi, su, si, spi,
      fu.reshape(B, 1), fi.reshape(B, 1), fpi.reshape(B, 1),
      time_since_prev_item, time_since_prev_user,
      uWe, uwt, user_b_ih.reshape(1, D), uWh, user_b_hh.reshape(1, D),
      iWe, iwt, item_b_ih.reshape(1, D), iWh, item_b_hh.reshape(1, D),
      predT, pb, tdw, tdb, initial_user_emb, initial_item_emb)

  du_r = jax.new_ref(dynamic_user_emb)
  di_r = jax.new_ref(dynamic_item_emb)
  fu_r = jax.new_ref(fu1)
  fi_r = jax.new_ref(fi1)
  _scatter()(du_r, di_r, fu_r, fi_r, user_id, item_id, uu, ui)
  new_du = jax.freeze(du_r)
  new_di = jax.freeze(di_r)
  new_fu = jax.freeze(fu_r).reshape(NUM_USERS, 1)
  new_fi = jax.freeze(fi_r).reshape(NUM_ITEMS, 1)

  return (item_pred, item_target, uu, ue, ui, ie,
          new_du, new_di, new_fu, new_fi)
